# baseline (speedup 1.0000x reference)
import functools

import jax
import jax.numpy as jnp
from jax import lax
from jax.experimental import pallas as pl
from jax.experimental.pallas import tpu as pltpu
from jax.experimental.pallas import tpu_sc as plsc

_NC = 2
_NS = 16


def _sc_gather_rows(idx, table, ch):
    B = idx.shape[0]
    E = table.shape[1]
    NW = _NC * _NS
    b_per_w = B // NW
    n_ch = b_per_w // ch
    mesh = plsc.VectorSubcoreMesh(core_axis_name="c", subcore_axis_name="s")

    @functools.partial(
        pl.kernel,
        out_type=jax.ShapeDtypeStruct((B, E), jnp.float32),
        mesh=mesh,
        scratch_types=[
            pltpu.VMEM((b_per_w,), jnp.int32),
            pltpu.VMEM((ch, E), jnp.float32),
            pltpu.VMEM((ch, E), jnp.float32),
            pltpu.SemaphoreType.DMA,
            pltpu.SemaphoreType.DMA,
            pltpu.SemaphoreType.DMA,
            pltpu.SemaphoreType.DMA,
        ],
    )
    def gather_kernel(idx_hbm, tab_hbm, out_hbm, idx_v, buf0, buf1,
                      g0, g1, s0, s1):
        wid = lax.axis_index("s") * _NC + lax.axis_index("c")
        base = wid * b_per_w
        pltpu.sync_copy(idx_hbm.at[pl.ds(base, b_per_w)], idx_v)

        bufs = (buf0, buf1)
        gsems = (g0, g1)
        ssems = (s0, s1)

        def start_gather(c):
            b = c & 1
            return pltpu.async_copy(
                tab_hbm.at[idx_v.at[pl.ds(c * ch, ch)]], bufs[b], gsems[b])

        gat = [None, None]
        scat = [None, None]
        gat[0] = start_gather(0)
        for c in range(n_ch):
            b = c & 1
            nb = (c + 1) & 1
            gat[b].wait()
            if c + 1 < n_ch:
                if scat[nb] is not None:
                    scat[nb].wait()
                gat[nb] = start_gather(c + 1)
            scat[b] = pltpu.make_async_copy(
                bufs[b], out_hbm.at[pl.ds(base + c * ch, ch)], ssems[b])
            scat[b].start()
        scat[0].wait()
        if n_ch > 1:
            scat[1].wait()

    return gather_kernel(idx, table)


def _tc_matmul(A, emd, bb):
    N, K = A.shape
    D = emd.shape[1]

    def body(a_ref, e_ref, o_ref):
        o_ref[...] = jnp.dot(a_ref[...], e_ref[...],
                             preferred_element_type=jnp.float32)

    return pl.pallas_call(
        body,
        grid=(N // bb,),
        in_specs=[
            pl.BlockSpec((bb, K), lambda i: (i, 0)),
            pl.BlockSpec((K, D), lambda i: (0, 0)),
        ],
        out_specs=pl.BlockSpec((bb, D), lambda i: (i, 0)),
        out_shape=jax.ShapeDtypeStruct((N, D), jnp.float32),
    )(A, emd)


def _tc_score(Ug, emd, iemb, bb):
    B, E = Ug.shape
    D = emd.shape[1]

    def body(ug_ref, e_ref, ie_ref, o_ref):
        pu = jnp.dot(ug_ref[...], e_ref[...],
                     preferred_element_type=jnp.float32)
        s = jnp.sum(pu * ie_ref[...], axis=1)
        o_ref[...] = jax.nn.sigmoid(s)

    return pl.pallas_call(
        body,
        grid=(B // bb,),
        in_specs=[
            pl.BlockSpec((bb, E), lambda i: (i, 0)),
            pl.BlockSpec((E, D), lambda i: (0, 0)),
            pl.BlockSpec((bb, D), lambda i: (i, 0)),
        ],
        out_specs=pl.BlockSpec((bb,), lambda i: (i,)),
        out_shape=jax.ShapeDtypeStruct((B,), jnp.float32),
    )(Ug, emd, iemb)


def kernel(users, items, enti_emd, UEnet, IEnet):
    emd_pad = jnp.pad(enti_emd, ((0, 0), (0, 128 - enti_emd.shape[1])))
    Ug = _sc_gather_rows(users, UEnet, ch=8)
    item_emd = _tc_matmul(IEnet, emd_pad, bb=512)
    items_emb = _sc_gather_rows(items, item_emd, ch=128)
    return _tc_score(Ug, emd_pad, items_emb, bb=256)
